# Initial kernel scaffold; baseline (speedup 1.0000x reference)
import jax
import jax.numpy as jnp
from jax.experimental import pallas as pl


def kernel(x, w_router, w1, b1, w2, b2, rng):
    raise NotImplementedError("write your pallas kernel here")


# speedup vs baseline: 1.1487x; 1.1487x over previous
import functools
import math

import jax
import jax.numpy as jnp
from jax.experimental import pallas as pl
from jax.experimental.pallas import tpu as pltpu

NUM_EXPERTS = 64
D_MODEL = 1024
D_FF = 1024
NUM_TOKENS = 2048
CAP = 32


def _moe_body(idx_ref, wts_ref, x_ref, w1_ref, b1_ref, w2_ref, b2_ref,
              out_ref, xe_ref):
    e = pl.program_id(0)

    @pl.when(e == 0)
    def _init():
        out_ref[...] = jnp.zeros_like(out_ref)

    for c in range(CAP):
        t = idx_ref[e, c]
        ts = jnp.where(t >= NUM_TOKENS, 0, t)
        xe_ref[pl.ds(c, 1), :] = x_ref[pl.ds(ts, 1), :]

    h = jnp.maximum(
        jnp.dot(xe_ref[...], w1_ref[0], preferred_element_type=jnp.float32)
        + b1_ref[0], 0.0)
    y = (jnp.dot(h, w2_ref[0], preferred_element_type=jnp.float32)
         + b2_ref[0])

    for c in range(CAP):
        t = idx_ref[e, c]

        @pl.when(t < NUM_TOKENS)
        def _store():
            out_ref[pl.ds(t, 1), :] = y[c:c + 1, :] * wts_ref[e, c]


@jax.jit
def kernel(x, w_router, w1, b1, w2, b2, rng):
    T, D = x.shape
    E = NUM_EXPERTS

    logits = x @ w_router
    top_e = jnp.argmax(logits, axis=-1).astype(jnp.int32)
    mx = jnp.max(logits, axis=-1)
    wt = 1.0 / jnp.sum(jnp.exp(logits - mx[:, None]), axis=-1)

    onehot = (jnp.arange(E, dtype=jnp.int32)[None, :] == top_e[:, None])
    pos = jnp.cumsum(onehot.astype(jnp.int32), axis=0) - 1
    pos_t = jnp.take_along_axis(pos, top_e[:, None], axis=1)[:, 0]
    slot = jnp.where(pos_t < CAP, top_e * CAP + pos_t, E * CAP)
    tok_ids = jnp.arange(T, dtype=jnp.int32)
    slot_to_token = jnp.full((E * CAP,), T, jnp.int32).at[slot].set(
        tok_ids, mode="drop", unique_indices=True).reshape(E, CAP)
    slot_wt = jnp.zeros((E * CAP,), jnp.float32).at[slot].set(
        wt, mode="drop", unique_indices=True).reshape(E, CAP)

    grid_spec = pltpu.PrefetchScalarGridSpec(
        num_scalar_prefetch=2,
        grid=(E,),
        in_specs=[
            pl.BlockSpec((T, D), lambda e, *_: (0, 0)),
            pl.BlockSpec((1, D, D_FF), lambda e, *_: (e, 0, 0)),
            pl.BlockSpec((1, 1, D_FF), lambda e, *_: (e, 0, 0)),
            pl.BlockSpec((1, D_FF, D), lambda e, *_: (e, 0, 0)),
            pl.BlockSpec((1, 1, D), lambda e, *_: (e, 0, 0)),
        ],
        out_specs=pl.BlockSpec((T, D), lambda e, *_: (0, 0)),
        scratch_shapes=[pltpu.VMEM((CAP, D), jnp.float32)],
    )
    out = pl.pallas_call(
        _moe_body,
        grid_spec=grid_spec,
        out_shape=jax.ShapeDtypeStruct((T, D), x.dtype),
        compiler_params=pltpu.CompilerParams(
            dimension_semantics=("arbitrary",),
        ),
    )(slot_to_token, slot_wt, x, w1, b1.reshape(E, 1, D_FF), w2,
      b2.reshape(E, 1, D))
    return out
